# baseline (speedup 1.0000x reference)
import jax
import jax.numpy as jnp
from jax.experimental import pallas as pl
from jax.experimental.pallas import tpu as pltpu

N0 = 50000
E0 = 800000
D0 = 84
H0 = 840
G0 = 512

RBLK = 2048
NP = 51200
EBLK = 4096
EP = 802816
DP = 96
HP = 896
GP = 520
NRB = NP // RBLK
NEB = EP // EBLK


def _segsum_body(idx_ref, x_ref, acc_ref):
    @pl.when(pl.program_id(0) == 0)
    def _init():
        acc_ref[...] = jnp.zeros_like(acc_ref)

    def body(i, carry):
        s = idx_ref[0, i]
        d = idx_ref[1, i]
        acc_ref[pl.ds(d, 1), :] = acc_ref[pl.ds(d, 1), :] + x_ref[pl.ds(s, 1), :]
        return carry

    jax.lax.fori_loop(0, EBLK, body, 0, unroll=16)


def _segment_sum(xp, eidx):
    return pl.pallas_call(
        _segsum_body,
        grid=(NEB,),
        in_specs=[
            pl.BlockSpec((2, EBLK), lambda e: (0, e), memory_space=pltpu.SMEM),
            pl.BlockSpec((NP, DP), lambda e: (0, 0)),
        ],
        out_specs=pl.BlockSpec((NP, DP), lambda e: (0, 0)),
        out_shape=jax.ShapeDtypeStruct((NP, DP), jnp.float32),
    )(eidx, xp)


def _mlp_body(x_ref, a_ref, w_ref, b_ref, o_ref):
    z = jnp.dot(x_ref[...] + a_ref[...], w_ref[...],
                preferred_element_type=jnp.float32)
    o_ref[...] = jnp.maximum(z + b_ref[0:1, :], 0.0)


def _mlp(xp, agg, wt, b, dout):
    return pl.pallas_call(
        _mlp_body,
        grid=(NRB,),
        in_specs=[
            pl.BlockSpec((RBLK, DP), lambda r: (r, 0)),
            pl.BlockSpec((RBLK, DP), lambda r: (r, 0)),
            pl.BlockSpec((DP, dout), lambda r: (0, 0)),
            pl.BlockSpec((8, dout), lambda r: (0, 0)),
        ],
        out_specs=pl.BlockSpec((RBLK, dout), lambda r: (r, 0)),
        out_shape=jax.ShapeDtypeStruct((NP, dout), jnp.float32),
    )(xp, agg, wt, b)


def _pool_body(bv_ref, bs_ref, h_ref, mx_ref, sm_ref, ct_ref):
    @pl.when(pl.program_id(0) == 0)
    def _init():
        mx_ref[...] = jnp.full_like(mx_ref, -jnp.inf)
        sm_ref[...] = jnp.zeros_like(sm_ref)
        ct_ref[...] = jnp.zeros_like(ct_ref)

    seg = bv_ref[0, 0, :]
    iota = jax.lax.broadcasted_iota(jnp.int32, (RBLK, GP), 1)
    p = (seg[:, None] == iota).astype(jnp.float32)
    h = h_ref[...]
    sm_ref[...] += jnp.dot(p.T, h, preferred_element_type=jnp.float32)
    ct_ref[...] += jnp.dot(p.T, jnp.ones((RBLK, 128), jnp.float32),
                           preferred_element_type=jnp.float32)

    def body(i, carry):
        s = bs_ref[0, 0, i]
        mx_ref[pl.ds(s, 1), :] = jnp.maximum(mx_ref[pl.ds(s, 1), :],
                                             h_ref[pl.ds(i, 1), :])
        return carry

    jax.lax.fori_loop(0, RBLK, body, 0, unroll=16)


def _pool(bp3, h2):
    return pl.pallas_call(
        _pool_body,
        grid=(NRB,),
        in_specs=[
            pl.BlockSpec((1, 1, RBLK), lambda r: (r, 0, 0)),
            pl.BlockSpec((1, 1, RBLK), lambda r: (r, 0, 0),
                         memory_space=pltpu.SMEM),
            pl.BlockSpec((RBLK, HP), lambda r: (r, 0)),
        ],
        out_specs=[
            pl.BlockSpec((GP, HP), lambda r: (0, 0)),
            pl.BlockSpec((GP, HP), lambda r: (0, 0)),
            pl.BlockSpec((GP, 128), lambda r: (0, 0)),
        ],
        out_shape=[
            jax.ShapeDtypeStruct((GP, HP), jnp.float32),
            jax.ShapeDtypeStruct((GP, HP), jnp.float32),
            jax.ShapeDtypeStruct((GP, 128), jnp.float32),
        ],
    )(bp3, bp3, h2)


def _fc_body(mx_ref, sm_ref, ct_ref, w1_ref, b1_ref, w2_ref, b2_ref, o_ref):
    mx = mx_ref[0:G0, :]
    mx = jnp.where(mx > -1e30, mx, 0.0)
    cnt = jnp.maximum(ct_ref[0:G0, 0:1], 1.0)
    mean = sm_ref[0:G0, :] / cnt
    pooled = jnp.concatenate([mx, mean], axis=1)
    hfc = jnp.maximum(
        jnp.dot(pooled, w1_ref[...], preferred_element_type=jnp.float32)
        + b1_ref[0:1, :], 0.0)
    o_ref[...] = (jnp.dot(hfc, w2_ref[...], preferred_element_type=jnp.float32)
                  + b2_ref[0:1, :])


def _fc(mx, sm, ct, wg1t, bg1p, wg2t, bg2p):
    return pl.pallas_call(
        _fc_body,
        out_shape=jax.ShapeDtypeStruct((G0, 384), jnp.float32),
    )(mx, sm, ct, wg1t, bg1p, wg2t, bg2p)


def kernel(x, edge_index, batch, W1, b1, W2, b2, Wg1, bg1, Wg2, bg2):
    f32 = jnp.float32
    xp = jnp.zeros((NP, DP), f32).at[:N0, :D0].set(x)
    ei = jnp.full((2, EP), N0, jnp.int32).at[:, :E0].set(edge_index)
    bp = jnp.full((NP,), G0, jnp.int32).at[:N0].set(batch)
    bp3 = bp.reshape(NRB, 1, RBLK)

    w1t = jnp.zeros((DP, DP), f32).at[:D0, :D0].set(W1.T)
    b1p = jnp.zeros((8, DP), f32).at[:, :D0].set(b1[None, :])
    w2t = jnp.zeros((DP, HP), f32).at[:D0, :H0].set(W2.T)
    b2p = jnp.zeros((8, HP), f32).at[:, :H0].set(b2[None, :])

    wg1t = jnp.zeros((2 * HP, 1024), f32)
    wg1t = wg1t.at[:H0, :].set(Wg1[:, :H0].T)
    wg1t = wg1t.at[HP:HP + H0, :].set(Wg1[:, H0:].T)
    bg1p = jnp.broadcast_to(bg1[None, :], (8, 1024))
    wg2t = Wg2.T
    bg2p = jnp.broadcast_to(bg2[None, :], (8, 384))

    agg1 = _segment_sum(xp, ei)
    h = _mlp(xp, agg1, w1t, b1p, DP)
    agg2 = _segment_sum(h, ei)
    h2 = _mlp(h, agg2, w2t, b2p, HP)
    mx, sm, ct = _pool(bp3, h2)
    return _fc(mx, sm, ct, wg1t, bg1p, wg2t, bg2p)


# speedup vs baseline: 1.3690x; 1.3690x over previous
import functools

import jax
import jax.numpy as jnp
from jax import lax
from jax.experimental import pallas as pl
from jax.experimental.pallas import tpu as pltpu
from jax.experimental.pallas import tpu_sc as plsc

N0 = 50000
E0 = 800000
D0 = 84
H0 = 840
G0 = 512

RBLK = 2048
NP = 51200
EBLK = 4096
EP = 802816
DP = 128
HP = 896
GP = 520
NRB = NP // RBLK
NEB = EP // EBLK


NSC = 50176
ESC = 802816
TPW = 196
EPW = TPW * 128
NCH = 4
CREAL = 12544
CROWS = 12672
ZPS = CROWS // 16
CPS = CREAL // 16


def _segsum_sc_body(x_hbm, src_hbm, dst_hbm, zrows_hbm, out_hbm,
                    src_v, dst_v, idx_v, rows_v, acc_sh, sem):
    cid = lax.axis_index("c")
    sid = lax.axis_index("s")
    wid = sid * 2 + cid
    ebase = wid * EPW
    for ch in range(NCH):
        base = ch * CREAL
        pltpu.sync_copy(zrows_hbm, acc_sh.at[pl.ds(sid * ZPS, ZPS)])
        plsc.subcore_barrier()

        def tile(t, carry):
            off = ebase + t * 128
            pltpu.sync_copy(src_hbm.at[pl.ds(off, 128)], src_v)
            pltpu.sync_copy(dst_hbm.at[pl.ds(off, 128)], dst_v)
            for j in range(8):
                v = dst_v[pl.ds(j * 16, 16)]
                t16 = v - jnp.full((16,), base, jnp.int32)
                ok = (t16 >= jnp.zeros((16,), jnp.int32)) & (
                    t16 < jnp.full((16,), CREAL, jnp.int32))
                idx_v[pl.ds(j * 16, 16)] = jnp.where(
                    ok, t16, jnp.full((16,), CREAL, jnp.int32))
            pltpu.async_copy(x_hbm.at[src_v], rows_v, sem).wait()
            pltpu.sync_copy(rows_v, acc_sh.at[idx_v], add=True)
            return carry

        lax.fori_loop(0, TPW, tile, 0)
        plsc.subcore_barrier()
        pltpu.sync_copy(
            acc_sh.at[pl.ds(sid * CPS, CPS)],
            out_hbm.at[cid, pl.ds(base + sid * CPS, CPS)])
        plsc.subcore_barrier()


def _segment_sum_sc(x128, src, dst, zrows):
    mesh = plsc.VectorSubcoreMesh(core_axis_name="c", subcore_axis_name="s")
    fn = functools.partial(
        pl.kernel,
        mesh=mesh,
        out_type=jax.ShapeDtypeStruct((2, NSC, DP), jnp.float32),
        scratch_types=[
            pltpu.VMEM((128,), jnp.int32),
            pltpu.VMEM((128,), jnp.int32),
            pltpu.VMEM((128,), jnp.int32),
            pltpu.VMEM((128, DP), jnp.float32),
            pltpu.VMEM_SHARED((CROWS, DP), jnp.float32),
            pltpu.SemaphoreType.DMA,
        ],
    )(_segsum_sc_body)
    return fn(x128, src, dst, zrows)


def _mlp_body(x_ref, a_ref, b2_ref, w_ref, b_ref, o_ref):
    z = jnp.dot(x_ref[...] + a_ref[...] + b2_ref[...], w_ref[...],
                preferred_element_type=jnp.float32)
    o_ref[...] = jnp.maximum(z + b_ref[0:1, :], 0.0)


def _mlp(xp, agg_a, agg_b, wt, b, dout):
    return pl.pallas_call(
        _mlp_body,
        grid=(NRB,),
        in_specs=[
            pl.BlockSpec((RBLK, DP), lambda r: (r, 0)),
            pl.BlockSpec((RBLK, DP), lambda r: (r, 0)),
            pl.BlockSpec((RBLK, DP), lambda r: (r, 0)),
            pl.BlockSpec((DP, dout), lambda r: (0, 0)),
            pl.BlockSpec((8, dout), lambda r: (0, 0)),
        ],
        out_specs=pl.BlockSpec((RBLK, dout), lambda r: (r, 0)),
        out_shape=jax.ShapeDtypeStruct((NP, dout), jnp.float32),
    )(xp, agg_a, agg_b, wt, b)


def _pool_body(bv_ref, bs_ref, h_ref, mx_ref, sm_ref, ct_ref):
    @pl.when(pl.program_id(0) == 0)
    def _init():
        mx_ref[...] = jnp.full_like(mx_ref, -jnp.inf)
        sm_ref[...] = jnp.zeros_like(sm_ref)
        ct_ref[...] = jnp.zeros_like(ct_ref)

    seg = bv_ref[0, 0, :]
    iota = jax.lax.broadcasted_iota(jnp.int32, (RBLK, GP), 1)
    p = (seg[:, None] == iota).astype(jnp.float32)
    h = h_ref[...]
    sm_ref[...] += jnp.dot(p.T, h, preferred_element_type=jnp.float32)
    ct_ref[...] += jnp.dot(p.T, jnp.ones((RBLK, 128), jnp.float32),
                           preferred_element_type=jnp.float32)

    def body(i, carry):
        s = bs_ref[0, 0, i]
        mx_ref[pl.ds(s, 1), :] = jnp.maximum(mx_ref[pl.ds(s, 1), :],
                                             h_ref[pl.ds(i, 1), :])
        return carry

    jax.lax.fori_loop(0, RBLK, body, 0, unroll=8)


def _pool(bp3, h2):
    return pl.pallas_call(
        _pool_body,
        grid=(NRB,),
        in_specs=[
            pl.BlockSpec((1, 1, RBLK), lambda r: (r, 0, 0)),
            pl.BlockSpec((1, 1, RBLK), lambda r: (r, 0, 0),
                         memory_space=pltpu.SMEM),
            pl.BlockSpec((RBLK, HP), lambda r: (r, 0)),
        ],
        out_specs=[
            pl.BlockSpec((GP, HP), lambda r: (0, 0)),
            pl.BlockSpec((GP, HP), lambda r: (0, 0)),
            pl.BlockSpec((GP, 128), lambda r: (0, 0)),
        ],
        out_shape=[
            jax.ShapeDtypeStruct((GP, HP), jnp.float32),
            jax.ShapeDtypeStruct((GP, HP), jnp.float32),
            jax.ShapeDtypeStruct((GP, 128), jnp.float32),
        ],
    )(bp3, bp3, h2)


def _fc_body(mx_ref, sm_ref, ct_ref, w1_ref, b1_ref, w2_ref, b2_ref, o_ref):
    mx = mx_ref[0:G0, :]
    mx = jnp.where(mx > -1e30, mx, 0.0)
    cnt = jnp.maximum(ct_ref[0:G0, 0:1], 1.0)
    mean = sm_ref[0:G0, :] / cnt
    pooled = jnp.concatenate([mx, mean], axis=1)
    hfc = jnp.maximum(
        jnp.dot(pooled, w1_ref[...], preferred_element_type=jnp.float32)
        + b1_ref[0:1, :], 0.0)
    o_ref[...] = (jnp.dot(hfc, w2_ref[...], preferred_element_type=jnp.float32)
                  + b2_ref[0:1, :])


def _fc(mx, sm, ct, wg1t, bg1p, wg2t, bg2p):
    return pl.pallas_call(
        _fc_body,
        out_shape=jax.ShapeDtypeStruct((G0, 384), jnp.float32),
    )(mx, sm, ct, wg1t, bg1p, wg2t, bg2p)


def _join(o, c):
    return jnp.zeros((NP, DP), jnp.float32).at[:NSC].set(o[c])


def kernel(x, edge_index, batch, W1, b1, W2, b2, Wg1, bg1, Wg2, bg2):
    f32 = jnp.float32
    xp = jnp.zeros((NP, DP), f32).at[:N0, :D0].set(x)
    src = jnp.full((ESC,), N0, jnp.int32).at[:E0].set(edge_index[0])
    dst = jnp.full((ESC,), N0, jnp.int32).at[:E0].set(edge_index[1])
    zrows = jnp.zeros((ZPS, DP), f32)
    bp = jnp.full((NP,), G0, jnp.int32).at[:N0].set(batch)
    bp3 = bp.reshape(NRB, 1, RBLK)

    w1t = jnp.zeros((DP, DP), f32).at[:D0, :D0].set(W1.T)
    b1p = jnp.zeros((8, DP), f32).at[:, :D0].set(b1[None, :])
    w2t = jnp.zeros((DP, HP), f32).at[:D0, :H0].set(W2.T)
    b2p = jnp.zeros((8, HP), f32).at[:, :H0].set(b2[None, :])

    wg1t = jnp.zeros((2 * HP, 1024), f32)
    wg1t = wg1t.at[:H0, :].set(Wg1[:, :H0].T)
    wg1t = wg1t.at[HP:HP + H0, :].set(Wg1[:, H0:].T)
    bg1p = jnp.broadcast_to(bg1[None, :], (8, 1024))
    wg2t = Wg2.T
    bg2p = jnp.broadcast_to(bg2[None, :], (8, 384))

    o1 = _segment_sum_sc(xp[:NSC], src, dst, zrows)
    h = _mlp(xp, _join(o1, 0), _join(o1, 1), w1t, b1p, DP)
    o2 = _segment_sum_sc(h[:NSC], src, dst, zrows)
    h2 = _mlp(h, _join(o2, 0), _join(o2, 1), w2t, b2p, HP)
    mx, sm, ct = _pool(bp3, h2)
    return _fc(mx, sm, ct, wg1t, bg1p, wg2t, bg2p)
